# initial kernel scaffold (unmeasured)
import jax
import jax.numpy as jnp
from jax import lax
from jax.experimental import pallas as pl
from jax.experimental.pallas import tpu as pltpu

N_DEV = 4
HQ = 32
HG = 8
DH = 128
SQ = 1024
SKV = 1024
DM = 1024
SCALE = 0.08838834764831843


def kernel(x, Wq, K_ext, V_ext, Wo):
    my_b = lax.axis_index("i")
    Kb = jnp.transpose(
        lax.dynamic_slice(K_ext, (my_b, 0, 0, 0), (1, SKV, HQ, DH))[0], (1, 0, 2)
    )
    Vb = jnp.transpose(
        lax.dynamic_slice(V_ext, (my_b, 0, 0, 0), (1, SKV, HQ, DH))[0], (1, 0, 2)
    )

    def body(x_ref, wq_ref, k_ref, v_ref, wo_ref, out_ref,
             comm_ref, send_sems, recv_sems):
        my_pos = lax.axis_index("i")
        left = lax.rem(my_pos + N_DEV - 1, N_DEV)
        right = lax.rem(my_pos + 1, N_DEV)

        barrier_sem = pltpu.get_barrier_semaphore()
        for nbr in (left, right):
            pl.semaphore_signal(
                barrier_sem, inc=1,
                device_id=(nbr,), device_id_type=pl.DeviceIdType.MESH,
            )
        pl.semaphore_wait(barrier_sem, 2)

        comm_ref[0, 0, :, :] = wq_ref[:, :]
        comm_ref[0, 1, :, :] = wo_ref[:, :]

        xv = x_ref[0]

        qi = lax.broadcasted_iota(jnp.int32, (SQ, SKV), 0)
        kj = lax.broadcasted_iota(jnp.int32, (SQ, SKV), 1)
        mask = ((qi // 64) % 4) == ((kj // 64) % 4)

        acc = jnp.zeros((SQ, DM), jnp.float32)
        for t in range(N_DEV):
            rdma = None
            if t < N_DEV - 1:
                rdma = pltpu.make_async_remote_copy(
                    src_ref=comm_ref.at[t],
                    dst_ref=comm_ref.at[t + 1],
                    send_sem=send_sems.at[t],
                    recv_sem=recv_sems.at[t],
                    device_id=(right,),
                    device_id_type=pl.DeviceIdType.MESH,
                )
                rdma.start()

            g = lax.rem(my_pos + N_DEV - t, N_DEV)
            wq_g = comm_ref[t, 0]
            wo_g = comm_ref[t, 1]
            q = jnp.dot(xv, wq_g, preferred_element_type=jnp.float32)
            ctx_cols = []
            for hh in range(HG):
                qh = q[:, hh * DH:(hh + 1) * DH]
                kh = k_ref[g * HG + hh]
                s = lax.dot_general(
                    qh, kh, (((1,), (1,)), ((), ())),
                    preferred_element_type=jnp.float32,
                ) * SCALE
                s = jnp.where(mask, s, -1e9)
                m = jnp.max(s, axis=1, keepdims=True)
                e = jnp.exp(s - m)
                w = e / jnp.sum(e, axis=1, keepdims=True)
                ctx_cols.append(
                    jnp.dot(w, v_ref[g * HG + hh],
                            preferred_element_type=jnp.float32)
                )
            ctx = jnp.concatenate(ctx_cols, axis=1)
            acc = acc + jnp.dot(ctx, wo_g, preferred_element_type=jnp.float32)

            if rdma is not None:
                rdma.wait()
        out_ref[0, :, :] = acc

    return pl.pallas_call(
        body,
        out_shape=jax.ShapeDtypeStruct((1, SQ, DM), jnp.float32),
        in_specs=[pl.BlockSpec(memory_space=pltpu.VMEM)] * 5,
        out_specs=pl.BlockSpec(memory_space=pltpu.VMEM),
        scratch_shapes=[
            pltpu.VMEM((N_DEV, 2, DM, DM), jnp.float32),
            pltpu.SemaphoreType.DMA((N_DEV - 1,)),
            pltpu.SemaphoreType.DMA((N_DEV - 1,)),
        ],
        compiler_params=pltpu.CompilerParams(collective_id=0),
    )(x, Wq, Kb, Vb, Wo)


# baseline (device time: 364794 ns/iter reference)
import jax
import jax.numpy as jnp
from jax import lax
from jax.experimental import pallas as pl
from jax.experimental.pallas import tpu as pltpu

N_DEV = 4
HQ = 32
HG = 8
DH = 128
SQ = 1024
SKV = 1024
DM = 1024
SCALE = 0.08838834764831843


def kernel(x, Wq, K_ext, V_ext, Wo):
    my_b = lax.axis_index("i")
    Kb = jnp.transpose(
        lax.dynamic_slice(K_ext, (my_b, 0, 0, 0), (1, SKV, HQ, DH))[0], (1, 0, 2)
    )
    Vb = jnp.transpose(
        lax.dynamic_slice(V_ext, (my_b, 0, 0, 0), (1, SKV, HQ, DH))[0], (1, 0, 2)
    )

    def body(x_ref, wq_ref, k_ref, v_ref, wo_ref, out_ref,
             comm_ref, kbuf, vbuf, ctx_ref,
             send_sems, recv_sems, local_sems):
        my_pos = lax.axis_index("i")
        left = lax.rem(my_pos + N_DEV - 1, N_DEV)
        right = lax.rem(my_pos + 1, N_DEV)

        wq_dma = pltpu.make_async_copy(wq_ref, comm_ref.at[0, 0], local_sems.at[0])
        wo_dma = pltpu.make_async_copy(wo_ref, comm_ref.at[0, 1], local_sems.at[1])
        wq_dma.start()
        wo_dma.start()

        barrier_sem = pltpu.get_barrier_semaphore()
        for nbr in (left, right):
            pl.semaphore_signal(
                barrier_sem, inc=1,
                device_id=(nbr,), device_id_type=pl.DeviceIdType.MESH,
            )
        pl.semaphore_wait(barrier_sem, 2)

        xv = x_ref[0]

        qi = lax.broadcasted_iota(jnp.int32, (SQ, SKV), 0)
        kj = lax.broadcasted_iota(jnp.int32, (SQ, SKV), 1)
        mask = ((qi // 64) % 4) == ((kj // 64) % 4)

        wq_dma.wait()
        wo_dma.wait()

        for t in range(N_DEV):
            g = lax.rem(my_pos + N_DEV - t, N_DEV)
            slot = t % 3

            k_dma = pltpu.make_async_copy(
                k_ref.at[pl.ds(g * HG, HG)], kbuf, local_sems.at[2])
            v_dma = pltpu.make_async_copy(
                v_ref.at[pl.ds(g * HG, HG)], vbuf, local_sems.at[3])
            k_dma.start()
            v_dma.start()

            rdma = None
            if t < N_DEV - 1:
                rdma = pltpu.make_async_remote_copy(
                    src_ref=comm_ref.at[slot],
                    dst_ref=comm_ref.at[(t + 1) % 3],
                    send_sem=send_sems.at[t],
                    recv_sem=recv_sems.at[t],
                    device_id=(right,),
                    device_id_type=pl.DeviceIdType.MESH,
                )
                rdma.start()

            q = jnp.dot(xv, comm_ref[slot, 0], preferred_element_type=jnp.float32)
            k_dma.wait()
            v_dma.wait()
            for hh in range(HG):
                qh = q[:, hh * DH:(hh + 1) * DH]
                s = lax.dot_general(
                    qh, kbuf[hh], (((1,), (1,)), ((), ())),
                    preferred_element_type=jnp.float32,
                ) * SCALE
                s = jnp.where(mask, s, -1e9)
                m = jnp.max(s, axis=1, keepdims=True)
                e = jnp.exp(s - m)
                w = e / jnp.sum(e, axis=1, keepdims=True)
                ctx_ref[:, hh * DH:(hh + 1) * DH] = jnp.dot(
                    w, vbuf[hh], preferred_element_type=jnp.float32)

            part = jnp.dot(ctx_ref[:, :], comm_ref[slot, 1],
                           preferred_element_type=jnp.float32)
            if t == 0:
                out_ref[0, :, :] = part
            else:
                out_ref[0, :, :] = out_ref[0, :, :] + part

            if rdma is not None:
                rdma.wait()

    return pl.pallas_call(
        body,
        out_shape=jax.ShapeDtypeStruct((1, SQ, DM), jnp.float32),
        in_specs=[
            pl.BlockSpec(memory_space=pltpu.VMEM),
            pl.BlockSpec(memory_space=pl.ANY),
            pl.BlockSpec(memory_space=pl.ANY),
            pl.BlockSpec(memory_space=pl.ANY),
            pl.BlockSpec(memory_space=pl.ANY),
        ],
        out_specs=pl.BlockSpec(memory_space=pltpu.VMEM),
        scratch_shapes=[
            pltpu.VMEM((3, 2, DM, DM), jnp.float32),
            pltpu.VMEM((HG, SKV, DH), jnp.float32),
            pltpu.VMEM((HG, SKV, DH), jnp.float32),
            pltpu.VMEM((SQ, DM), jnp.float32),
            pltpu.SemaphoreType.DMA((N_DEV - 1,)),
            pltpu.SemaphoreType.DMA((N_DEV - 1,)),
            pltpu.SemaphoreType.DMA((4,)),
        ],
        compiler_params=pltpu.CompilerParams(
            collective_id=0, vmem_limit_bytes=60 * 1024 * 1024,
        ),
    )(x, Wq, Kb, Vb, Wo)


# device time: 211772 ns/iter; 1.7226x vs baseline; 1.7226x over previous
import jax
import jax.numpy as jnp
from jax import lax
from jax.experimental import pallas as pl
from jax.experimental.pallas import tpu as pltpu

N_DEV = 4
HQ = 32
HG = 8
DH = 128
SQ = 1024
SKV = 1024
DM = 1024
SCALE = 0.08838834764831843
BF = jnp.bfloat16


def kernel(x, Wq, K_ext, V_ext, Wo):
    my_b = lax.axis_index("i")
    x16 = x.astype(BF)
    wq16 = Wq.astype(BF)
    wo16 = Wo.astype(BF)
    Kb = jnp.transpose(
        lax.dynamic_slice(K_ext, (my_b, 0, 0, 0), (1, SKV, HQ, DH))[0], (1, 0, 2)
    ).astype(BF)
    Vb = jnp.transpose(
        lax.dynamic_slice(V_ext, (my_b, 0, 0, 0), (1, SKV, HQ, DH))[0], (1, 0, 2)
    ).astype(BF)

    def body(x_ref, wq_ref, k_ref, v_ref, wo_ref, out_ref,
             comm_ref, kbuf, vbuf, ctx_ref,
             send_sems, recv_sems, local_sems):
        my_pos = lax.axis_index("i")
        left = lax.rem(my_pos + N_DEV - 1, N_DEV)
        right = lax.rem(my_pos + 1, N_DEV)

        wq_dma = pltpu.make_async_copy(wq_ref, comm_ref.at[0, 0], local_sems.at[0])
        wo_dma = pltpu.make_async_copy(wo_ref, comm_ref.at[0, 1], local_sems.at[1])
        wq_dma.start()
        wo_dma.start()

        barrier_sem = pltpu.get_barrier_semaphore()
        for nbr in (left, right):
            pl.semaphore_signal(
                barrier_sem, inc=1,
                device_id=(nbr,), device_id_type=pl.DeviceIdType.MESH,
            )
        pl.semaphore_wait(barrier_sem, 2)

        xv = x_ref[0]

        qi = lax.broadcasted_iota(jnp.int32, (SQ, SKV), 0)
        kj = lax.broadcasted_iota(jnp.int32, (SQ, SKV), 1)
        mask = ((qi // 64) % 4) == ((kj // 64) % 4)

        wq_dma.wait()
        wo_dma.wait()

        for t in range(N_DEV):
            g = lax.rem(my_pos + N_DEV - t, N_DEV)
            slot = t % 3

            k_dma = pltpu.make_async_copy(
                k_ref.at[pl.ds(g * HG, HG)], kbuf, local_sems.at[2])
            v_dma = pltpu.make_async_copy(
                v_ref.at[pl.ds(g * HG, HG)], vbuf, local_sems.at[3])
            k_dma.start()
            v_dma.start()

            rdma = None
            if t < N_DEV - 1:
                rdma = pltpu.make_async_remote_copy(
                    src_ref=comm_ref.at[slot],
                    dst_ref=comm_ref.at[(t + 1) % 3],
                    send_sem=send_sems.at[t],
                    recv_sem=recv_sems.at[t],
                    device_id=(right,),
                    device_id_type=pl.DeviceIdType.MESH,
                )
                rdma.start()

            q = jnp.dot(xv, comm_ref[slot, 0],
                        preferred_element_type=jnp.float32).astype(BF)
            k_dma.wait()
            v_dma.wait()
            for hh in range(HG):
                qh = q[:, hh * DH:(hh + 1) * DH]
                s = lax.dot_general(
                    qh, kbuf[hh], (((1,), (1,)), ((), ())),
                    preferred_element_type=jnp.float32,
                ) * SCALE
                s = jnp.where(mask, s, -1e9)
                m = jnp.max(s, axis=1, keepdims=True)
                e = jnp.exp(s - m)
                w = (e / jnp.sum(e, axis=1, keepdims=True)).astype(BF)
                ctx_ref[:, hh * DH:(hh + 1) * DH] = jnp.dot(
                    w, vbuf[hh], preferred_element_type=jnp.float32).astype(BF)

            part = jnp.dot(ctx_ref[:, :], comm_ref[slot, 1],
                           preferred_element_type=jnp.float32)
            if t == 0:
                out_ref[0, :, :] = part
            else:
                out_ref[0, :, :] = out_ref[0, :, :] + part

            if rdma is not None:
                rdma.wait()

    return pl.pallas_call(
        body,
        out_shape=jax.ShapeDtypeStruct((1, SQ, DM), jnp.float32),
        in_specs=[
            pl.BlockSpec(memory_space=pltpu.VMEM),
            pl.BlockSpec(memory_space=pl.ANY),
            pl.BlockSpec(memory_space=pl.ANY),
            pl.BlockSpec(memory_space=pl.ANY),
            pl.BlockSpec(memory_space=pl.ANY),
        ],
        out_specs=pl.BlockSpec(memory_space=pltpu.VMEM),
        scratch_shapes=[
            pltpu.VMEM((3, 2, DM, DM), BF),
            pltpu.VMEM((HG, SKV, DH), BF),
            pltpu.VMEM((HG, SKV, DH), BF),
            pltpu.VMEM((SQ, DM), BF),
            pltpu.SemaphoreType.DMA((N_DEV - 1,)),
            pltpu.SemaphoreType.DMA((N_DEV - 1,)),
            pltpu.SemaphoreType.DMA((4,)),
        ],
        compiler_params=pltpu.CompilerParams(
            collective_id=0, vmem_limit_bytes=60 * 1024 * 1024,
        ),
    )(x16, wq16, Kb, Vb, wo16)


# device time: 138356 ns/iter; 2.6366x vs baseline; 1.5306x over previous
import jax
import jax.numpy as jnp
from jax import lax
from jax.experimental import pallas as pl
from jax.experimental.pallas import tpu as pltpu

N_DEV = 4
HQ = 32
HG = 8
DH = 128
SQ = 1024
SKV = 1024
DM = 1024
SCALE = 0.08838834764831843
BF = jnp.bfloat16


def kernel(x, Wq, K_ext, V_ext, Wo):
    x16 = x.astype(BF)
    wq16 = Wq.astype(BF)
    wo16 = Wo.astype(BF)

    def body(x_ref, wq_ref, k_ref, v_ref, wo_ref, out_ref,
             comm_ref, kbuf, vbuf, ctx_ref,
             send_sems, recv_a, recv_b, w_sems, kv_sems):
        my_pos = lax.axis_index("i")
        my_b = my_pos
        left = lax.rem(my_pos + N_DEV - 1, N_DEV)
        right = lax.rem(my_pos + 1, N_DEV)

        wq_dma = pltpu.make_async_copy(wq_ref, comm_ref.at[0, 0], w_sems.at[0])
        wo_dma = pltpu.make_async_copy(wo_ref, comm_ref.at[0, 1], w_sems.at[1])
        wq_dma.start()
        wo_dma.start()

        gs = [my_pos, left, right, lax.rem(my_pos + 2, N_DEV)]

        def kv_dma(s):
            g = gs[s]
            return (
                pltpu.make_async_copy(
                    k_ref.at[my_b, :, pl.ds(g * HG, HG), :],
                    kbuf.at[s % 2], kv_sems.at[0, s % 2]),
                pltpu.make_async_copy(
                    v_ref.at[my_b, :, pl.ds(g * HG, HG), :],
                    vbuf.at[s % 2], kv_sems.at[1, s % 2]),
            )

        k0, v0 = kv_dma(0)
        k0.start()
        v0.start()

        barrier_sem = pltpu.get_barrier_semaphore()
        for nbr in (left, right):
            pl.semaphore_signal(
                barrier_sem, inc=1,
                device_id=(nbr,), device_id_type=pl.DeviceIdType.MESH,
            )
        pl.semaphore_wait(barrier_sem, 2)

        wq_dma.wait()
        wo_dma.wait()

        a_right = pltpu.make_async_remote_copy(
            src_ref=comm_ref.at[0], dst_ref=comm_ref.at[1],
            send_sem=send_sems.at[0], recv_sem=recv_a.at[0],
            device_id=(right,), device_id_type=pl.DeviceIdType.MESH,
        )
        a_left = pltpu.make_async_remote_copy(
            src_ref=comm_ref.at[0], dst_ref=comm_ref.at[2],
            send_sem=send_sems.at[1], recv_sem=recv_a.at[1],
            device_id=(left,), device_id_type=pl.DeviceIdType.MESH,
        )
        a_right.start()
        a_left.start()

        b_right = pltpu.make_async_remote_copy(
            src_ref=comm_ref.at[1, 0], dst_ref=comm_ref.at[3, 0],
            send_sem=send_sems.at[2], recv_sem=recv_b.at[0],
            device_id=(right,), device_id_type=pl.DeviceIdType.MESH,
        )
        b_left = pltpu.make_async_remote_copy(
            src_ref=comm_ref.at[2, 1], dst_ref=comm_ref.at[3, 1],
            send_sem=send_sems.at[3], recv_sem=recv_b.at[1],
            device_id=(left,), device_id_type=pl.DeviceIdType.MESH,
        )

        xv = x_ref[0]

        qi = lax.broadcasted_iota(jnp.int32, (SQ, SKV), 0)
        kj = lax.broadcasted_iota(jnp.int32, (SQ, SKV), 1)
        mask = ((qi // 64) % 4) == ((kj // 64) % 4)

        def compute(s, slot):
            kv_dma_wait = s
            q = jnp.dot(xv, comm_ref[slot, 0],
                        preferred_element_type=jnp.float32)
            pltpu.make_async_copy(
                kbuf.at[s % 2], kbuf.at[s % 2], kv_sems.at[0, s % 2]).wait()
            pltpu.make_async_copy(
                vbuf.at[s % 2], vbuf.at[s % 2], kv_sems.at[1, s % 2]).wait()
            if s < 3:
                kn, vn = kv_dma(s + 1)
                kn.start()
                vn.start()
            for hh in range(HG):
                qh = q[:, hh * DH:(hh + 1) * DH]
                sc = lax.dot_general(
                    qh, kbuf[s % 2, :, hh, :], (((1,), (1,)), ((), ())),
                    preferred_element_type=jnp.float32,
                ) * SCALE
                sc = jnp.where(mask, sc, -1e9)
                mx = jnp.max(sc, axis=1, keepdims=True)
                e = jnp.exp(sc - mx)
                w = (e / jnp.sum(e, axis=1, keepdims=True)).astype(BF)
                ctx_ref[:, hh * DH:(hh + 1) * DH] = jnp.dot(
                    w, vbuf[s % 2, :, hh, :].astype(BF),
                    preferred_element_type=jnp.float32).astype(BF)
            part = jnp.dot(ctx_ref[:, :], comm_ref[slot, 1],
                           preferred_element_type=jnp.float32)
            if s == 0:
                out_ref[0, :, :] = part
            else:
                out_ref[0, :, :] = out_ref[0, :, :] + part

        compute(0, 0)

        a_right.wait_recv()
        b_right.start()
        compute(1, 1)

        a_left.wait_recv()
        b_left.start()
        compute(2, 2)

        b_right.wait_recv()
        b_left.wait_recv()
        compute(3, 3)

        a_right.wait_send()
        a_left.wait_send()
        b_right.wait_send()
        b_left.wait_send()

    return pl.pallas_call(
        body,
        out_shape=jax.ShapeDtypeStruct((1, SQ, DM), jnp.float32),
        in_specs=[
            pl.BlockSpec(memory_space=pltpu.VMEM),
            pl.BlockSpec(memory_space=pl.ANY),
            pl.BlockSpec(memory_space=pl.ANY),
            pl.BlockSpec(memory_space=pl.ANY),
            pl.BlockSpec(memory_space=pl.ANY),
        ],
        out_specs=pl.BlockSpec(memory_space=pltpu.VMEM),
        scratch_shapes=[
            pltpu.VMEM((4, 2, DM, DM), BF),
            pltpu.VMEM((2, SKV, HG, DH), jnp.float32),
            pltpu.VMEM((2, SKV, HG, DH), jnp.float32),
            pltpu.VMEM((SQ, DM), BF),
            pltpu.SemaphoreType.DMA((4,)),
            pltpu.SemaphoreType.DMA((2,)),
            pltpu.SemaphoreType.DMA((2,)),
            pltpu.SemaphoreType.DMA((2,)),
            pltpu.SemaphoreType.DMA((2, 2)),
        ],
        compiler_params=pltpu.CompilerParams(
            collective_id=0, vmem_limit_bytes=60 * 1024 * 1024,
        ),
    )(x16, wq16, K_ext, V_ext, wo16)


# device time: 129482 ns/iter; 2.8173x vs baseline; 1.0685x over previous
import jax
import jax.numpy as jnp
from jax import lax
from jax.experimental import pallas as pl
from jax.experimental.pallas import tpu as pltpu

N_DEV = 4
HQ = 32
HG = 8
DH = 128
SQ = 1024
SKV = 1024
DM = 1024
NP = 4
BLK = 64
PER = SQ // NP
SCALE = 0.08838834764831843
BF = jnp.bfloat16

_PK = [(p, k) for p in range(NP) for k in range(NP)]


def kernel(x, Wq, K_ext, V_ext, Wo):
    x16 = x.astype(BF)
    wq16 = Wq.astype(BF)
    wo16 = Wo.astype(BF)

    def body(x_ref, wq_ref, k_ref, v_ref, wo_ref, out_ref,
             comm_ref, kbuf, vbuf, ctx_ref, acc_ref,
             send_sems, recv_a, recv_b, w_sems, kv_sems):
        my_pos = lax.axis_index("i")
        my_b = my_pos
        left = lax.rem(my_pos + N_DEV - 1, N_DEV)
        right = lax.rem(my_pos + 1, N_DEV)

        wq_dma = pltpu.make_async_copy(wq_ref, comm_ref.at[0, 0], w_sems.at[0])
        wo_dma = pltpu.make_async_copy(wo_ref, comm_ref.at[0, 1], w_sems.at[1])
        wq_dma.start()
        wo_dma.start()

        gs = [my_pos, left, right, lax.rem(my_pos + 2, N_DEV)]

        def kv_start(s):
            g = gs[s]
            for (p, k) in _PK:
                pltpu.make_async_copy(
                    k_ref.at[my_b, pl.ds(BLK * (p + 4 * k), BLK),
                             pl.ds(g * HG, HG), :],
                    kbuf.at[s % 2, pl.ds(PER * p + BLK * k, BLK)],
                    kv_sems.at[0, s % 2]).start()
                pltpu.make_async_copy(
                    v_ref.at[my_b, pl.ds(BLK * (p + 4 * k), BLK),
                             pl.ds(g * HG, HG), :],
                    vbuf.at[s % 2, pl.ds(PER * p + BLK * k, BLK)],
                    kv_sems.at[1, s % 2]).start()

        def kv_wait(s):
            pltpu.make_async_copy(
                kbuf.at[s % 2], kbuf.at[s % 2], kv_sems.at[0, s % 2]).wait()
            pltpu.make_async_copy(
                vbuf.at[s % 2], vbuf.at[s % 2], kv_sems.at[1, s % 2]).wait()

        kv_start(0)

        barrier_sem = pltpu.get_barrier_semaphore()
        for nbr in (left, right):
            pl.semaphore_signal(
                barrier_sem, inc=1,
                device_id=(nbr,), device_id_type=pl.DeviceIdType.MESH,
            )
        pl.semaphore_wait(barrier_sem, 2)

        wq_dma.wait()
        wo_dma.wait()

        a_right = pltpu.make_async_remote_copy(
            src_ref=comm_ref.at[0], dst_ref=comm_ref.at[1],
            send_sem=send_sems.at[0], recv_sem=recv_a.at[0],
            device_id=(right,), device_id_type=pl.DeviceIdType.MESH,
        )
        a_left = pltpu.make_async_remote_copy(
            src_ref=comm_ref.at[0], dst_ref=comm_ref.at[2],
            send_sem=send_sems.at[1], recv_sem=recv_a.at[1],
            device_id=(left,), device_id_type=pl.DeviceIdType.MESH,
        )
        a_right.start()
        a_left.start()

        b_right = pltpu.make_async_remote_copy(
            src_ref=comm_ref.at[1, 0], dst_ref=comm_ref.at[3, 0],
            send_sem=send_sems.at[2], recv_sem=recv_b.at[0],
            device_id=(right,), device_id_type=pl.DeviceIdType.MESH,
        )
        b_left = pltpu.make_async_remote_copy(
            src_ref=comm_ref.at[2, 1], dst_ref=comm_ref.at[3, 1],
            send_sem=send_sems.at[3], recv_sem=recv_b.at[1],
            device_id=(left,), device_id_type=pl.DeviceIdType.MESH,
        )

        xp = jnp.concatenate(
            [x_ref[0, pl.ds(BLK * (p + 4 * k), BLK), :] for (p, k) in _PK],
            axis=0)

        def compute(s, slot):
            q = jnp.dot(xp, comm_ref[slot, 0],
                        preferred_element_type=jnp.float32).astype(BF)
            kv_wait(s)
            if s < 3:
                kv_start(s + 1)
            for hh in range(HG):
                qh = q[:, hh * DH:(hh + 1) * DH].reshape(NP, PER, DH)
                kh = kbuf[s % 2, :, hh, :].astype(BF).reshape(NP, PER, DH)
                sc = lax.dot_general(
                    qh, kh, (((2,), (2,)), ((0,), (0,))),
                    preferred_element_type=jnp.float32,
                ) * SCALE
                mx = jnp.max(sc, axis=2, keepdims=True)
                e = jnp.exp(sc - mx)
                w = (e / jnp.sum(e, axis=2, keepdims=True)).astype(BF)
                vh = vbuf[s % 2, :, hh, :].astype(BF).reshape(NP, PER, DH)
                ctxh = lax.dot_general(
                    w, vh, (((2,), (1,)), ((0,), (0,))),
                    preferred_element_type=jnp.float32)
                ctx_ref[:, hh * DH:(hh + 1) * DH] = (
                    ctxh.reshape(SQ, DH).astype(BF))
            part = jnp.dot(ctx_ref[:, :], comm_ref[slot, 1],
                           preferred_element_type=jnp.float32)
            if s == 0:
                acc_ref[:, :] = part
            elif s < 3:
                acc_ref[:, :] = acc_ref[:, :] + part
            else:
                full = acc_ref[:, :] + part
                for (p, k) in _PK:
                    out_ref[0, pl.ds(BLK * (p + 4 * k), BLK), :] = (
                        full[PER * p + BLK * k:PER * p + BLK * (k + 1), :])

        compute(0, 0)

        a_right.wait_recv()
        b_right.start()
        compute(1, 1)

        a_left.wait_recv()
        b_left.start()
        compute(2, 2)

        b_right.wait_recv()
        b_left.wait_recv()
        compute(3, 3)

        a_right.wait_send()
        a_left.wait_send()
        b_right.wait_send()
        b_left.wait_send()

    return pl.pallas_call(
        body,
        out_shape=jax.ShapeDtypeStruct((1, SQ, DM), jnp.float32),
        in_specs=[
            pl.BlockSpec(memory_space=pltpu.VMEM),
            pl.BlockSpec(memory_space=pl.ANY),
            pl.BlockSpec(memory_space=pl.ANY),
            pl.BlockSpec(memory_space=pl.ANY),
            pl.BlockSpec(memory_space=pl.ANY),
        ],
        out_specs=pl.BlockSpec(memory_space=pltpu.VMEM),
        scratch_shapes=[
            pltpu.VMEM((4, 2, DM, DM), BF),
            pltpu.VMEM((2, SKV, HG, DH), jnp.float32),
            pltpu.VMEM((2, SKV, HG, DH), jnp.float32),
            pltpu.VMEM((SQ, DM), BF),
            pltpu.VMEM((SQ, DM), jnp.float32),
            pltpu.SemaphoreType.DMA((4,)),
            pltpu.SemaphoreType.DMA((2,)),
            pltpu.SemaphoreType.DMA((2,)),
            pltpu.SemaphoreType.DMA((2,)),
            pltpu.SemaphoreType.DMA((2, 2)),
        ],
        compiler_params=pltpu.CompilerParams(
            collective_id=0, vmem_limit_bytes=60 * 1024 * 1024,
        ),
    )(x16, wq16, K_ext, V_ext, wo16)
